# baseline (speedup 1.0000x reference)
import functools

import jax
import jax.numpy as jnp
from jax import lax
from jax.experimental import pallas as pl
from jax.experimental.pallas import tpu as pltpu
from jax.experimental.pallas import tpu_sc as plsc

N_ATOMS = 10000
N_BONDS = 320000
HIDDEN = 128
MAX_NB = 32
DEPTH = 6

NW = 32
ATOMS_PAD = 10240
ATOMS_PER_W = ATOMS_PAD // NW
A_CHUNK = 1
A_NCHUNK = ATOMS_PER_W // A_CHUNK
A_NBUF = 4
A_SLAB = 32
N_BONDS_H = N_BONDS // 2
BONDS_PER_W = N_BONDS_H // NW
B_CHUNK = 40
B_NCHUNK = BONDS_PER_W // B_CHUNK


def _make_nei_kernel(width):
    ncg = width // 16
    mesh = plsc.VectorSubcoreMesh(
        core_axis_name="c", subcore_axis_name="s", num_cores=2, num_subcores=16)

    @functools.partial(
        pl.kernel,
        out_type=jax.ShapeDtypeStruct((ATOMS_PAD, width), jnp.float32),
        mesh=mesh,
        scratch_types=(
            [pltpu.VMEM((A_NCHUNK, A_CHUNK * MAX_NB), jnp.int32)]
            + [pltpu.VMEM((A_CHUNK * MAX_NB, width), jnp.float32)
               for _ in range(A_NBUF)]
            + [pltpu.VMEM((A_SLAB, width), jnp.float32)
               for _ in range(2)]
            + [pltpu.SemaphoreType.DMA for _ in range(A_NBUF)]
            + [pltpu.SemaphoreType.DMA for _ in range(2)]
        ),
    )
    def nei_kernel(msg_hbm, msg2_hbm, a2b_hbm, out_hbm, idx_v, *bufs):
        msgs = (msg_hbm, msg2_hbm)
        rows = bufs[0:A_NBUF]
        slabs = bufs[A_NBUF:A_NBUF + 2]
        gsems = bufs[A_NBUF + 2:2 * A_NBUF + 2]
        wsems = bufs[2 * A_NBUF + 2:2 * A_NBUF + 4]
        wid = lax.axis_index("s") * 2 + lax.axis_index("c")
        base_atom = wid * ATOMS_PER_W
        pltpu.sync_copy(a2b_hbm.at[wid], idx_v)

        def start(c, k):
            pltpu.async_copy(msgs[k % 2].at[idx_v.at[c]], rows[k], gsems[k])

        def wait(c, k):
            pltpu.make_async_copy(
                msgs[k % 2].at[idx_v.at[c]], rows[k], gsems[k]).wait()

        def slab_hbm(first_atom):
            off = pl.multiple_of(base_atom + first_atom, A_SLAB)
            return out_hbm.at[pl.ds(off, A_SLAB)]

        def compute(c, k):
            rows_v = rows[k]
            slab_row = lax.rem(c, A_SLAB)
            parity = lax.rem(lax.div(c, A_SLAB), 2)

            @pl.when((slab_row == 0) & (c >= 2 * A_SLAB))
            def _drain():
                for p in range(2):
                    @pl.when(parity == p)
                    def _d(p=p):
                        pltpu.make_async_copy(
                            slabs[p], slab_hbm(c - 2 * A_SLAB), wsems[p]).wait()

            def nb_body(q, carry):
                out = carry
                for u in range(4):
                    row = q * 4 + u
                    out = tuple(
                        out[cg] + rows_v[row, pl.ds(cg * 16, 16)]
                        for cg in range(ncg)
                    )
                return out

            acc = lax.fori_loop(
                0, MAX_NB // 4, nb_body,
                tuple(jnp.zeros((16,), jnp.float32) for _ in range(ncg)),
            )
            for p in range(2):
                @pl.when(parity == p)
                def _store(p=p):
                    for cg in range(ncg):
                        slabs[p][slab_row, pl.ds(cg * 16, 16)] = acc[cg]

            @pl.when(slab_row == A_SLAB - 1)
            def _flush():
                for p in range(2):
                    @pl.when(parity == p)
                    def _w(p=p):
                        pltpu.async_copy(
                            slabs[p], slab_hbm(c - (A_SLAB - 1)), wsems[p])

        for k in range(A_NBUF):
            start(k, k)

        def ring_body(c4, _):
            c = c4 * A_NBUF
            for k in range(A_NBUF):
                wait(c + k, k)
                compute(c + k, k)

                @pl.when(c + k + A_NBUF < A_NCHUNK)
                def _next(k=k):
                    start(c + k + A_NBUF, k)

            return _

        lax.fori_loop(0, A_NCHUNK // A_NBUF, ring_body, 0)
        for p in range(2):
            pltpu.make_async_copy(
                slabs[p],
                slab_hbm(A_NCHUNK - (2 - p) * A_SLAB), wsems[p]).wait()

    return nei_kernel


def _make_comb_kernel(width):
    ncg = width // 16
    mesh = plsc.VectorSubcoreMesh(
        core_axis_name="c", subcore_axis_name="s", num_cores=2, num_subcores=16)

    @functools.partial(
        pl.kernel,
        out_type=jax.ShapeDtypeStruct((N_BONDS_H, width), jnp.float32),
        mesh=mesh,
        scratch_types=[
            pltpu.VMEM((B_NCHUNK, B_CHUNK), jnp.int32),
            pltpu.VMEM((B_NCHUNK, B_CHUNK), jnp.int32),
            pltpu.VMEM((B_CHUNK, width), jnp.float32),
            pltpu.VMEM((B_CHUNK, width), jnp.float32),
            pltpu.VMEM((B_CHUNK, width), jnp.float32),
            pltpu.VMEM((B_CHUNK, width), jnp.float32),
            pltpu.SemaphoreType.DMA,
            pltpu.SemaphoreType.DMA,
        ],
    )
    def comb_kernel(nei_hbm, msg_hbm, b2a_hbm, b2revb_hbm, out_hbm,
                    idxa_v, idxr_v, nrows0_v, nrows1_v, mrows0_v, mrows1_v,
                    sem0, sem1):
        wid = lax.axis_index("s") * 2 + lax.axis_index("c")
        base_bond = wid * BONDS_PER_W
        pltpu.sync_copy(b2a_hbm.at[wid], idxa_v)
        pltpu.sync_copy(b2revb_hbm.at[wid], idxr_v)

        def start(c, nrows_v, mrows_v, sem):
            pltpu.async_copy(nei_hbm.at[idxa_v.at[c]], nrows_v, sem)
            pltpu.async_copy(msg_hbm.at[idxr_v.at[c]], mrows_v, sem)

        def wait(c, nrows_v, mrows_v, sem):
            pltpu.make_async_copy(nei_hbm.at[idxa_v.at[c]], nrows_v, sem).wait()
            pltpu.make_async_copy(msg_hbm.at[idxr_v.at[c]], mrows_v, sem).wait()

        def compute(c, nrows_v, mrows_v):
            def row_body(r, _):
                for cg in range(ncg):
                    sl = pl.ds(cg * 16, 16)
                    nrows_v[r, sl] = nrows_v[r, sl] - mrows_v[r, sl]
                return _

            lax.fori_loop(0, B_CHUNK, row_body, 0)
            pltpu.sync_copy(
                nrows_v, out_hbm.at[pl.ds(base_bond + c * B_CHUNK, B_CHUNK)])

        start(0, nrows0_v, mrows0_v, sem0)

        def pair_body(c2, _):
            c = c2 * 2
            wait(c, nrows0_v, mrows0_v, sem0)
            start(c + 1, nrows1_v, mrows1_v, sem1)
            compute(c, nrows0_v, mrows0_v)
            wait(c + 1, nrows1_v, mrows1_v, sem1)

            @pl.when(c2 + 1 < B_NCHUNK // 2)
            def _start_next():
                start(c + 2, nrows0_v, mrows0_v, sem0)

            compute(c + 1, nrows1_v, mrows1_v)
            return _

        lax.fori_loop(0, B_NCHUNK // 2, pair_body, 0)

    return comb_kernel


_MM_ROWS = 1280
_MM_NBLK = N_BONDS_H // _MM_ROWS


def _mm_body(x_ref, w_ref, o_ref):
    o_ref[...] = jnp.maximum(
        jnp.dot(x_ref[...], w_ref[0], preferred_element_type=jnp.float32), 0.0)


def _mm_body_alias(big_ref, x_ref, w_ref, o_ref):
    del big_ref
    _mm_body(x_ref, w_ref, o_ref)


def _matmul_relu_half0(x, wt_stack, in_width):
    x_map = (lambda i, j: (i, 0)) if in_width == HIDDEN else (lambda i, j: (i, j))
    return pl.pallas_call(
        _mm_body,
        grid=(_MM_NBLK, 3),
        in_specs=[
            pl.BlockSpec((_MM_ROWS, HIDDEN), x_map),
            pl.BlockSpec((1, HIDDEN, HIDDEN), lambda i, j: (j, 0, 0)),
        ],
        out_specs=pl.BlockSpec((_MM_ROWS, HIDDEN), lambda i, j: (i, j)),
        out_shape=jax.ShapeDtypeStruct((N_BONDS_H, 3 * HIDDEN), jnp.float32),
        compiler_params=pltpu.CompilerParams(
            dimension_semantics=("parallel", "arbitrary")),
    )(x, wt_stack)


def _matmul_relu_half1(big, x, wt_stack, in_width):
    x_map = (lambda i, j: (i, 0)) if in_width == HIDDEN else (lambda i, j: (i, j))
    return pl.pallas_call(
        _mm_body_alias,
        grid=(_MM_NBLK, 3),
        in_specs=[
            pl.BlockSpec((_MM_ROWS, HIDDEN), lambda i, j: (i, j)),
            pl.BlockSpec((_MM_ROWS, HIDDEN), x_map),
            pl.BlockSpec((1, HIDDEN, HIDDEN), lambda i, j: (j, 0, 0)),
        ],
        out_specs=pl.BlockSpec(
            (_MM_ROWS, HIDDEN), lambda i, j: (i + _MM_NBLK, j)),
        out_shape=jax.ShapeDtypeStruct((N_BONDS, 3 * HIDDEN), jnp.float32),
        input_output_aliases={0: 0},
        compiler_params=pltpu.CompilerParams(
            dimension_semantics=("parallel", "arbitrary")),
    )(big, x, wt_stack)


_make_nei_kernel = functools.lru_cache(maxsize=None)(_make_nei_kernel)
_make_comb_kernel = functools.lru_cache(maxsize=None)(_make_comb_kernel)


def kernel(f_atoms, f_bonds, a2b, a2a, b2a, b2revb, W_h_q, W_h_k, W_h_v):
    del f_atoms, a2a
    a2b = a2b.astype(jnp.int32)
    b2a = b2a.astype(jnp.int32)
    b2revb = b2revb.astype(jnp.int32)

    a2b_pad = jnp.zeros((ATOMS_PAD, MAX_NB), jnp.int32).at[:N_ATOMS].set(a2b)
    a2b_r = a2b_pad.reshape(NW, A_NCHUNK, A_CHUNK * MAX_NB)
    b2a_h = [b2a[h * N_BONDS_H:(h + 1) * N_BONDS_H].reshape(NW, B_NCHUNK, B_CHUNK)
             for h in range(2)]
    b2revb_h = [b2revb[h * N_BONDS_H:(h + 1) * N_BONDS_H].reshape(
        NW, B_NCHUNK, B_CHUNK) for h in range(2)]
    wt_stack = jnp.stack([W_h_q.T, W_h_k.T, W_h_v.T])

    def one_iter(msg, width):
        nei = _make_nei_kernel(width)(msg, msg, a2b_r)
        comb = _make_comb_kernel(width)
        new_a = comb(nei, msg, b2a_h[0], b2revb_h[0])
        new_b = comb(nei, msg, b2a_h[1], b2revb_h[1])
        big = _matmul_relu_half0(new_a, wt_stack, width)
        big2 = _matmul_relu_half0(new_b, wt_stack, width)
        return jnp.concatenate([big, big2], axis=0)

    msg = one_iter(f_bonds, HIDDEN)
    for _ in range(DEPTH - 2):
        msg = one_iter(msg, 3 * HIDDEN)

    return (msg[:, :HIDDEN], msg[:, HIDDEN:2 * HIDDEN], msg[:, 2 * HIDDEN:])


# speedup vs baseline: 1.2559x; 1.2559x over previous
import functools

import jax
import jax.numpy as jnp
import numpy as np
from jax import lax
from jax.experimental import pallas as pl
from jax.experimental.pallas import tpu as pltpu
from jax.experimental.pallas import tpu_sc as plsc

N_ATOMS = 10000
N_BONDS = 320000
HIDDEN = 128
MAX_NB = 32
DEPTH = 6

NW = 32
ATOMS_PAD = 10240
ATOMS_PER_W = ATOMS_PAD // NW
A_CHUNK = 1
A_NCHUNK = ATOMS_PER_W // A_CHUNK
A_NBUF = 4
A_SLAB = 32
BONDS_PER_W = N_BONDS // NW
B_CHUNK = 40
B_NCHUNK = BONDS_PER_W // B_CHUNK


def _make_nei_kernel(width):
    ncg = width // 16
    mesh = plsc.VectorSubcoreMesh(
        core_axis_name="c", subcore_axis_name="s", num_cores=2, num_subcores=16)

    @functools.partial(
        pl.kernel,
        out_type=jax.ShapeDtypeStruct((ATOMS_PAD, width), jnp.float32),
        mesh=mesh,
        scratch_types=(
            [pltpu.VMEM((A_NCHUNK, A_CHUNK * MAX_NB), jnp.int32)]
            + [pltpu.VMEM((A_CHUNK * MAX_NB, width), jnp.float32)
               for _ in range(A_NBUF)]
            + [pltpu.VMEM((A_SLAB, width), jnp.float32)
               for _ in range(2)]
            + [pltpu.SemaphoreType.DMA for _ in range(A_NBUF)]
            + [pltpu.SemaphoreType.DMA for _ in range(2)]
        ),
    )
    def nei_kernel(msg_hbm, msg2_hbm, a2b_hbm, out_hbm, idx_v, *bufs):
        msgs = (msg_hbm, msg2_hbm)
        rows = bufs[0:A_NBUF]
        slabs = bufs[A_NBUF:A_NBUF + 2]
        gsems = bufs[A_NBUF + 2:2 * A_NBUF + 2]
        wsems = bufs[2 * A_NBUF + 2:2 * A_NBUF + 4]
        wid = lax.axis_index("s") * 2 + lax.axis_index("c")
        base_atom = wid * ATOMS_PER_W
        pltpu.sync_copy(a2b_hbm.at[wid], idx_v)

        def start(c, k):
            pltpu.async_copy(msgs[k % 2].at[idx_v.at[c]], rows[k], gsems[k])

        def wait(c, k):
            pltpu.make_async_copy(
                msgs[k % 2].at[idx_v.at[c]], rows[k], gsems[k]).wait()

        def slab_hbm(first_atom):
            off = pl.multiple_of(base_atom + first_atom, A_SLAB)
            return out_hbm.at[pl.ds(off, A_SLAB)]

        def compute(c, k):
            rows_v = rows[k]
            slab_row = lax.rem(c, A_SLAB)
            parity = lax.rem(lax.div(c, A_SLAB), 2)

            @pl.when((slab_row == 0) & (c >= 2 * A_SLAB))
            def _drain():
                for p in range(2):
                    @pl.when(parity == p)
                    def _d(p=p):
                        pltpu.make_async_copy(
                            slabs[p], slab_hbm(c - 2 * A_SLAB), wsems[p]).wait()

            def nb_body(q, carry):
                out = carry
                for u in range(4):
                    row = q * 4 + u
                    out = tuple(
                        out[cg] + rows_v[row, pl.ds(cg * 16, 16)]
                        for cg in range(ncg)
                    )
                return out

            acc = lax.fori_loop(
                0, MAX_NB // 4, nb_body,
                tuple(jnp.zeros((16,), jnp.float32) for _ in range(ncg)),
            )
            for p in range(2):
                @pl.when(parity == p)
                def _store(p=p):
                    for cg in range(ncg):
                        slabs[p][slab_row, pl.ds(cg * 16, 16)] = acc[cg]

            @pl.when(slab_row == A_SLAB - 1)
            def _flush():
                for p in range(2):
                    @pl.when(parity == p)
                    def _w(p=p):
                        pltpu.async_copy(
                            slabs[p], slab_hbm(c - (A_SLAB - 1)), wsems[p])

        for k in range(A_NBUF):
            start(k, k)

        def ring_body(c4, _):
            c = c4 * A_NBUF
            for k in range(A_NBUF):
                wait(c + k, k)
                compute(c + k, k)

                @pl.when(c + k + A_NBUF < A_NCHUNK)
                def _next(k=k):
                    start(c + k + A_NBUF, k)

            return _

        lax.fori_loop(0, A_NCHUNK // A_NBUF, ring_body, 0)
        for p in range(2):
            pltpu.make_async_copy(
                slabs[p],
                slab_hbm(A_NCHUNK - (2 - p) * A_SLAB), wsems[p]).wait()

    return nei_kernel


def _make_comb_kernel(width):
    ncg = width // 16
    mesh = plsc.VectorSubcoreMesh(
        core_axis_name="c", subcore_axis_name="s", num_cores=2, num_subcores=16)

    @functools.partial(
        pl.kernel,
        out_type=jax.ShapeDtypeStruct((N_BONDS, width), jnp.float32),
        mesh=mesh,
        scratch_types=[
            pltpu.VMEM((B_NCHUNK, B_CHUNK), jnp.int32),
            pltpu.VMEM((B_NCHUNK, B_CHUNK), jnp.int32),
            pltpu.VMEM((B_CHUNK, width), jnp.float32),
            pltpu.VMEM((B_CHUNK, width), jnp.float32),
            pltpu.VMEM((B_CHUNK, width), jnp.float32),
            pltpu.VMEM((B_CHUNK, width), jnp.float32),
            pltpu.SemaphoreType.DMA,
            pltpu.SemaphoreType.DMA,
        ],
    )
    def comb_kernel(nei_hbm, msg_hbm, b2a_hbm, b2revb_hbm, out_hbm,
                    idxa_v, idxr_v, nrows0_v, nrows1_v, mrows0_v, mrows1_v,
                    sem0, sem1):
        wid = lax.axis_index("s") * 2 + lax.axis_index("c")
        base_bond = wid * BONDS_PER_W
        pltpu.sync_copy(b2a_hbm.at[wid], idxa_v)
        pltpu.sync_copy(b2revb_hbm.at[wid], idxr_v)

        def start(c, nrows_v, mrows_v, sem):
            pltpu.async_copy(nei_hbm.at[idxa_v.at[c]], nrows_v, sem)
            pltpu.async_copy(msg_hbm.at[idxr_v.at[c]], mrows_v, sem)

        def wait(c, nrows_v, mrows_v, sem):
            pltpu.make_async_copy(nei_hbm.at[idxa_v.at[c]], nrows_v, sem).wait()
            pltpu.make_async_copy(msg_hbm.at[idxr_v.at[c]], mrows_v, sem).wait()

        def compute(c, nrows_v, mrows_v):
            def row_body(r, _):
                for cg in range(ncg):
                    sl = pl.ds(cg * 16, 16)
                    nrows_v[r, sl] = nrows_v[r, sl] - mrows_v[r, sl]
                return _

            lax.fori_loop(0, B_CHUNK, row_body, 0)
            pltpu.sync_copy(
                nrows_v, out_hbm.at[pl.ds(base_bond + c * B_CHUNK, B_CHUNK)])

        start(0, nrows0_v, mrows0_v, sem0)

        def pair_body(c2, _):
            c = c2 * 2
            wait(c, nrows0_v, mrows0_v, sem0)
            start(c + 1, nrows1_v, mrows1_v, sem1)
            compute(c, nrows0_v, mrows0_v)
            wait(c + 1, nrows1_v, mrows1_v, sem1)

            @pl.when(c2 + 1 < B_NCHUNK // 2)
            def _start_next():
                start(c + 2, nrows0_v, mrows0_v, sem0)

            compute(c + 1, nrows1_v, mrows1_v)
            return _

        lax.fori_loop(0, B_NCHUNK // 2, pair_body, 0)

    return comb_kernel


_MM_ROWS = 1280
_MM_NBLK = N_BONDS // _MM_ROWS


def _make_mm_body(in_width):
    def _mm_body(x_ref, w_ref, o_ref):
        xb = x_ref[...]
        for j in range(3):
            xj = xb if in_width == HIDDEN else xb[:, j * HIDDEN:(j + 1) * HIDDEN]
            o_ref[:, j * HIDDEN:(j + 1) * HIDDEN] = jnp.maximum(
                jnp.dot(xj, w_ref[j], preferred_element_type=jnp.float32), 0.0)
    return _mm_body


def _matmul_relu(x, wt_stack, in_width):
    return pl.pallas_call(
        _make_mm_body(in_width),
        grid=(_MM_NBLK,),
        in_specs=[
            pl.BlockSpec((_MM_ROWS, in_width), lambda i: (i, 0)),
            pl.BlockSpec((3, HIDDEN, HIDDEN), lambda i: (0, 0, 0)),
        ],
        out_specs=pl.BlockSpec((_MM_ROWS, 3 * HIDDEN), lambda i: (i, 0)),
        out_shape=jax.ShapeDtypeStruct((N_BONDS, 3 * HIDDEN), jnp.float32),
        compiler_params=pltpu.CompilerParams(
            dimension_semantics=("parallel",)),
    )(x, wt_stack)


_make_nei_kernel = functools.lru_cache(maxsize=None)(_make_nei_kernel)
_make_comb_kernel = functools.lru_cache(maxsize=None)(_make_comb_kernel)


def kernel(f_atoms, f_bonds, a2b, a2a, b2a, b2revb, W_h_q, W_h_k, W_h_v):
    del f_atoms, a2a
    a2b = a2b.astype(jnp.int32)
    b2a = b2a.astype(jnp.int32)
    b2revb = b2revb.astype(jnp.int32)

    a2b_pad = jnp.zeros((ATOMS_PAD, MAX_NB), jnp.int32).at[:N_ATOMS].set(a2b)
    a2b_r = a2b_pad.reshape(NW, A_NCHUNK, A_CHUNK * MAX_NB)
    b2a_r = b2a.reshape(NW, B_NCHUNK, B_CHUNK)
    b2revb_r = b2revb.reshape(NW, B_NCHUNK, B_CHUNK)
    wt_stack = jnp.stack([W_h_q.T, W_h_k.T, W_h_v.T])

    def one_iter(msg, width):
        nei = _make_nei_kernel(width)(msg, msg, a2b_r)
        new = _make_comb_kernel(width)(nei, msg, b2a_r, b2revb_r)
        return _matmul_relu(new, wt_stack, width)

    msg = one_iter(f_bonds, HIDDEN)
    for _ in range(DEPTH - 2):
        msg = one_iter(msg, 3 * HIDDEN)

    return (msg[:, :HIDDEN], msg[:, HIDDEN:2 * HIDDEN], msg[:, 2 * HIDDEN:])
